# baseline (speedup 1.0000x reference)
import functools

import jax
import jax.numpy as jnp
from jax import lax
from jax.experimental import pallas as pl
from jax.experimental.pallas import tpu as pltpu
from jax.experimental.pallas import tpu_sc as plsc

NPAD = 100352
NPT = NPAD // 16
NF = NPAD // 8
CH = 128
K = 6
BNODE = 2048
GRID = NPAD // BNODE
BR = 1792
_F32 = jnp.float32

_MESH = plsc.VectorSubcoreMesh(core_axis_name="c", subcore_axis_name="s")
_SC_PARAMS = pltpu.CompilerParams(use_tc_tiling_on_sc=False)


KD = 8


def _deg_kernel(nbursts):
    @functools.partial(
        pl.kernel,
        out_type=jax.ShapeDtypeStruct((2 * NPAD,), _F32),
        mesh=_MESH,
        scratch_types=[
            pltpu.VMEM((KD, CH), jnp.int32),
            pltpu.VMEM((KD, CH), _F32),
            pltpu.VMEM_SHARED((NPAD,), _F32),
            pltpu.SemaphoreType.DMA,
        ],
        compiler_params=_SC_PARAMS,
    )
    def deg(dst2d, ones_h, zeros1, out, didx, ones_v, acc, sem):
        c = lax.axis_index("c")
        s = lax.axis_index("s")
        off = pl.multiple_of(s * NPT, 8)
        pltpu.sync_copy(zeros1, acc.at[pl.ds(off, NPT)])
        pltpu.sync_copy(ones_h, ones_v)
        plsc.subcore_barrier()
        row0 = (c * 16 + s) * (nbursts * KD)

        def body(i, carry):
            rb = pl.multiple_of(row0 + i * KD, 8)
            pltpu.sync_copy(dst2d.at[pl.ds(rb, KD)], didx)
            descs = [
                pltpu.async_copy(ones_v.at[j], acc.at[didx.at[j]], sem, add=True)
                for j in range(KD)
            ]
            for d in descs:
                d.wait()
            return carry

        lax.fori_loop(0, nbursts, body, 0)
        plsc.subcore_barrier()
        dst_off = pl.multiple_of(c * NPAD + off, 8)
        pltpu.sync_copy(acc.at[pl.ds(off, NPT)], out.at[pl.ds(dst_off, NPT)])

    return deg


def _agg_kernel(nbursts):
    assert nbursts % 2 == 0
    npairs = nbursts // 2

    @functools.partial(
        pl.kernel,
        out_type=jax.ShapeDtypeStruct((2 * NPAD, 16), _F32),
        mesh=_MESH,
        scratch_types=[
            pltpu.VMEM((2, 2 * K, CH), jnp.int32),
            pltpu.VMEM((2, K, CH, 16), _F32),
            pltpu.VMEM_SHARED((NPAD, 16), _F32),
            pltpu.SemaphoreType.DMA,
            pltpu.SemaphoreType.DMA,
            pltpu.SemaphoreType.DMA,
            pltpu.SemaphoreType.DMA,
        ],
        compiler_params=_SC_PARAMS,
    )
    def agg(table, eidx2d, zrows, out, eidx, rows, acc,
            sem_g0, sem_g1, sem_s0, sem_s1):
        c = lax.axis_index("c")
        s = lax.axis_index("s")
        off = pl.multiple_of(s * NPT, 8)
        pltpu.sync_copy(zrows, acc.at[pl.ds(off, NPT)])
        plsc.subcore_barrier()
        row0 = (c * 16 + s) * (nbursts * K)
        sem_g = (sem_g0, sem_g1)
        sem_s = (sem_s0, sem_s1)

        def fire_gathers(i, b):
            rb = pl.multiple_of(2 * (row0 + i * K), 4)
            pltpu.sync_copy(eidx2d.at[pl.ds(rb, 2 * K)], eidx.at[b])
            for j in range(K):
                pltpu.async_copy(
                    table.at[eidx.at[b, 2 * j]], rows.at[b, j], sem_g[b])

        def drain_gathers(b):
            for j in range(K):
                pltpu.make_async_copy(
                    table.at[eidx.at[b, 2 * j]], rows.at[b, j], sem_g[b]).wait()

        def fire_scatters(b):
            for j in range(K):
                pltpu.async_copy(
                    rows.at[b, j], acc.at[eidx.at[b, 2 * j + 1]], sem_s[b],
                    add=True)

        def drain_scatters(b):
            for j in range(K):
                pltpu.make_async_copy(
                    rows.at[b, j], acc.at[eidx.at[b, 2 * j + 1]], sem_s[b]).wait()

        fire_gathers(0, 0)

        def body(t, carry):
            a = 2 * t

            @pl.when(t > 0)
            def _():
                drain_scatters(1)
            fire_gathers(a + 1, 1)
            drain_gathers(0)
            fire_scatters(0)
            drain_scatters(0)

            @pl.when(t + 1 < npairs)
            def _():
                fire_gathers(a + 2, 0)
            drain_gathers(1)
            fire_scatters(1)
            return carry

        lax.fori_loop(0, npairs, body, 0)
        drain_scatters(1)
        plsc.subcore_barrier()
        dst_off = pl.multiple_of(c * NPAD + off, 8)
        pltpu.sync_copy(acc.at[pl.ds(off, NPT)], out.at[pl.ds(dst_off, NPT)])

    return agg


def _t0_body(d0_ref, d1_ref, dv_ref):
    dv_ref[...] = lax.rsqrt(d0_ref[...] + d1_ref[...] + 1.0)


def _t1_body(x_ref, dvw_ref, hs_ref):
    hs_ref[...] = x_ref[...] * dvw_ref[...]


def _t2_body(a0, a1, hs, dvw, w, b, out):
    z = (a0[...] + a1[...] + hs[...]) * dvw[...]
    u = jnp.dot(z, w[...], preferred_element_type=_F32,
                precision=lax.Precision.HIGHEST) + b[...]
    out[...] = jnp.tanh(u) * dvw[...]


def _t3_body(a0, a1, hs, dvw, w, b, s16, out):
    z = (a0[...] + a1[...] + hs[...]) * dvw[...]
    u = jnp.dot(z, w[...], preferred_element_type=_F32,
                precision=lax.Precision.HIGHEST) + b[...]
    ss = jnp.dot(u * u, s16[...], preferred_element_type=_F32,
                 precision=lax.Precision.HIGHEST)
    h = jnp.tanh(u / jnp.maximum(jnp.sqrt(ss), 1e-12))
    out[...] = h * dvw[...]


def _t4_body(a0, a1, hs, dvw, w3l, w3r, b3l, b3r, wcl, wcr, bc, s16, out):
    hp = lax.Precision.HIGHEST
    z = (a0[...] + a1[...] + hs[...]) * dvw[...]
    ul = jnp.dot(z, w3l[...], preferred_element_type=_F32, precision=hp) + b3l[...]
    ur = jnp.dot(z, w3r[...], preferred_element_type=_F32, precision=hp) + b3r[...]
    ss = jnp.dot(ul * ul + ur * ur, s16[...], preferred_element_type=_F32,
                 precision=hp)
    inv = 1.0 / jnp.maximum(jnp.sqrt(ss), 1e-12)
    h3l = ul * inv
    h3r = ur * inv
    v = (jnp.dot(h3l, wcl[...], preferred_element_type=_F32, precision=hp)
         + jnp.dot(h3r, wcr[...], preferred_element_type=_F32, precision=hp)
         + bc[...])
    ss2 = jnp.dot(v * v, s16[...], preferred_element_type=_F32, precision=hp)
    out[...] = v / jnp.maximum(jnp.sqrt(ss2), 1e-12)


def _spec(rows, imap):
    return pl.BlockSpec((rows, 128), imap)


def _cspec(shape):
    return pl.BlockSpec(shape, lambda i: (0, 0))


def _blockdiag(w16):
    return jnp.kron(jnp.eye(8, dtype=_F32), w16)


def kernel(x, edge_index, W1, b1, W2, b2, W3, b3, Wc, bc):
    n = x.shape[0]
    e = edge_index.shape[1]
    burst_edges = 32 * K * CH
    nbursts = -(-e // burst_edges)
    nbursts += nbursts % 2
    epad = nbursts * burst_edges
    erows = epad // CH
    nbursts_d = epad // (32 * KD * CH)
    assert nbursts_d * 32 * KD * CH == epad

    src = edge_index[0].astype(jnp.int32)
    dst = edge_index[1].astype(jnp.int32)
    pad = epad - e
    src2d = jnp.concatenate([src, jnp.zeros((pad,), jnp.int32)]).reshape(erows, CH)
    dst2d = jnp.concatenate([dst, jnp.full((pad,), n, jnp.int32)]).reshape(erows, CH)

    zrows = jnp.zeros((NPT, 16), _F32)
    zeros1 = jnp.zeros((NPT,), _F32)
    ones_h = jnp.ones((KD, CH), _F32)

    w1b = _blockdiag(jnp.zeros((16, 16), _F32).at[:3, :6].set(W1))
    b1b = jnp.tile(jnp.zeros((1, 16), _F32).at[0, :6].set(b1), (1, 8))
    w2b = _blockdiag(jnp.zeros((16, 16), _F32).at[:6, :12].set(W2))
    b2b = jnp.tile(jnp.zeros((1, 16), _F32).at[0, :12].set(b2), (1, 8))
    w3lb = _blockdiag(jnp.zeros((16, 16), _F32).at[:12, :12].set(W3[:, :12]))
    w3rb = _blockdiag(jnp.zeros((16, 16), _F32).at[:12, :12].set(W3[:, 12:]))
    b3lb = jnp.tile(jnp.zeros((1, 16), _F32).at[0, :12].set(b3[:12]), (1, 8))
    b3rb = jnp.tile(jnp.zeros((1, 16), _F32).at[0, :12].set(b3[12:]), (1, 8))
    wclb = _blockdiag(jnp.zeros((16, 16), _F32).at[:12, :13].set(Wc[:12]))
    wcrb = _blockdiag(jnp.zeros((16, 16), _F32).at[:12, :13].set(Wc[12:]))
    bcb = jnp.tile(jnp.zeros((1, 16), _F32).at[0, :13].set(bc), (1, 8))
    s16b = _blockdiag(jnp.ones((16, 16), _F32))

    deg = _deg_kernel(nbursts_d)(dst2d, ones_h, zeros1)
    deg2d = deg.reshape(2 * NPAD // 128, 128)

    dv_lanes = pl.pallas_call(
        _t0_body, grid=(GRID,),
        in_specs=[
            pl.BlockSpec((16, 128), lambda i: (i, 0)),
            pl.BlockSpec((16, 128), lambda i: (i + GRID, 0)),
        ],
        out_specs=pl.BlockSpec((16, 128), lambda i: (i, 0)),
        out_shape=jax.ShapeDtypeStruct((NPAD // 128, 128), _F32),
    )(deg2d, deg2d)

    dvwf = jnp.broadcast_to(dv_lanes.reshape(NPAD, 1), (NPAD, 16)).reshape(NF, 128)
    x16f = jnp.zeros((NPAD, 16), _F32).at[:n, :3].set(x).reshape(NF, 128)

    hs1f = pl.pallas_call(
        _t1_body, grid=(NF // BR,),
        in_specs=[_spec(BR, lambda i: (i, 0))] * 2,
        out_specs=_spec(BR, lambda i: (i, 0)),
        out_shape=jax.ShapeDtypeStruct((NF, 128), _F32),
    )(x16f, dvwf)

    agg = _agg_kernel(nbursts)
    nfb = NF // BR

    def dense(body, aggf, hsf, consts):
        cspecs = [_cspec(c.shape) for c in consts]
        return pl.pallas_call(
            body, grid=(nfb,),
            in_specs=[
                _spec(BR, lambda i: (i, 0)),
                _spec(BR, lambda i: (i + nfb, 0)),
                _spec(BR, lambda i: (i, 0)),
                _spec(BR, lambda i: (i, 0)),
            ] + cspecs,
            out_specs=_spec(BR, lambda i: (i, 0)),
            out_shape=jax.ShapeDtypeStruct((NF, 128), _F32),
        )(aggf, aggf, hsf, dvwf, *consts)

    eidx2d = jnp.stack([src2d, dst2d], axis=1).reshape(2 * erows, CH)

    a1f = agg(hs1f.reshape(NPAD, 16), eidx2d, zrows).reshape(2 * NF, 128)
    hs2f = dense(_t2_body, a1f, hs1f, [w1b, b1b])

    a2f = agg(hs2f.reshape(NPAD, 16), eidx2d, zrows).reshape(2 * NF, 128)
    hs3f = dense(_t3_body, a2f, hs2f, [w2b, b2b, s16b])

    a3f = agg(hs3f.reshape(NPAD, 16), eidx2d, zrows).reshape(2 * NF, 128)
    outf = dense(_t4_body, a3f, hs3f,
                 [w3lb, w3rb, b3lb, b3rb, wclb, wcrb, bcb, s16b])

    return outf.reshape(NPAD, 16)[:n, :13]


# speedup vs baseline: 1.3072x; 1.3072x over previous
import functools

import jax
import jax.numpy as jnp
from jax import lax
from jax.experimental import pallas as pl
from jax.experimental.pallas import tpu as pltpu
from jax.experimental.pallas import tpu_sc as plsc

NPAD = 100352
NPT = NPAD // 16
NF = NPAD // 8
CH = 128
K = 4
BNODE = 2048
GRID = NPAD // BNODE
BR = 1792
_F32 = jnp.float32

_MESH = plsc.VectorSubcoreMesh(core_axis_name="c", subcore_axis_name="s")
_SC_PARAMS = pltpu.CompilerParams(use_tc_tiling_on_sc=False)


KD = 8


def _deg_kernel(nbursts):
    assert nbursts % 2 == 0
    npairs = nbursts // 2

    @functools.partial(
        pl.kernel,
        out_type=jax.ShapeDtypeStruct((2 * NPAD,), _F32),
        mesh=_MESH,
        scratch_types=[
            pltpu.VMEM((2, KD, CH), jnp.int32),
            pltpu.VMEM((KD, CH), _F32),
            pltpu.VMEM_SHARED((NPAD,), _F32),
            pltpu.SemaphoreType.DMA,
            pltpu.SemaphoreType.DMA,
        ],
        compiler_params=_SC_PARAMS,
    )
    def deg(dst2d, ones_h, zeros1, out, didx, ones_v, acc, sem0, sem1):
        c = lax.axis_index("c")
        s = lax.axis_index("s")
        off = pl.multiple_of(s * NPT, 8)
        pltpu.sync_copy(zeros1, acc.at[pl.ds(off, NPT)])
        pltpu.sync_copy(ones_h, ones_v)
        plsc.subcore_barrier()
        row0 = (c * 16 + s) * (nbursts * KD)
        sems = (sem0, sem1)

        def load_idx(i, b):
            rb = pl.multiple_of(row0 + i * KD, 8)
            pltpu.sync_copy(dst2d.at[pl.ds(rb, KD)], didx.at[b])

        def fire(b):
            for j in range(KD):
                pltpu.async_copy(ones_v.at[j], acc.at[didx.at[b, j]], sems[b],
                                 add=True)

        def drain(b):
            for j in range(KD):
                pltpu.make_async_copy(
                    ones_v.at[j], acc.at[didx.at[b, j]], sems[b]).wait()

        load_idx(0, 0)

        def body(t, carry):
            fire(0)

            @pl.when(t > 0)
            def _():
                drain(1)
            load_idx(2 * t + 1, 1)
            fire(1)
            drain(0)

            @pl.when(t + 1 < npairs)
            def _():
                load_idx(2 * t + 2, 0)
            return carry

        lax.fori_loop(0, npairs, body, 0)
        drain(1)
        plsc.subcore_barrier()
        dst_off = pl.multiple_of(c * NPAD + off, 8)
        pltpu.sync_copy(acc.at[pl.ds(off, NPT)], out.at[pl.ds(dst_off, NPT)])

    return deg


def _agg_kernel(nbursts):
    assert nbursts % 2 == 0
    npairs = nbursts // 2

    @functools.partial(
        pl.kernel,
        out_type=jax.ShapeDtypeStruct((2 * NPAD, 16), _F32),
        mesh=_MESH,
        scratch_types=[
            pltpu.VMEM((2, 2 * K, CH), jnp.int32),
            pltpu.VMEM((2, K, CH, 16), _F32),
            pltpu.VMEM_SHARED((NPAD, 16), _F32),
            pltpu.SemaphoreType.DMA,
            pltpu.SemaphoreType.DMA,
            pltpu.SemaphoreType.DMA,
            pltpu.SemaphoreType.DMA,
        ],
        compiler_params=_SC_PARAMS,
    )
    def agg(table, eidx2d, zrows, out, eidx, rows, acc,
            sem_g0, sem_g1, sem_s0, sem_s1):
        c = lax.axis_index("c")
        s = lax.axis_index("s")
        off = pl.multiple_of(s * NPT, 8)
        pltpu.sync_copy(zrows, acc.at[pl.ds(off, NPT)])
        plsc.subcore_barrier()
        row0 = (c * 16 + s) * (nbursts * K)
        sem_g = (sem_g0, sem_g1)
        sem_s = (sem_s0, sem_s1)

        def fire_gathers(i, b):
            rb = pl.multiple_of(2 * (row0 + i * K), 8)
            pltpu.sync_copy(eidx2d.at[pl.ds(rb, 2 * K)], eidx.at[b])
            for j in range(K):
                pltpu.async_copy(
                    table.at[eidx.at[b, 2 * j]], rows.at[b, j], sem_g[b])

        def drain_gathers(b):
            for j in range(K):
                pltpu.make_async_copy(
                    table.at[eidx.at[b, 2 * j]], rows.at[b, j], sem_g[b]).wait()

        def fire_scatters(b):
            for j in range(K):
                pltpu.async_copy(
                    rows.at[b, j], acc.at[eidx.at[b, 2 * j + 1]], sem_s[b],
                    add=True)

        def drain_scatters(b):
            for j in range(K):
                pltpu.make_async_copy(
                    rows.at[b, j], acc.at[eidx.at[b, 2 * j + 1]], sem_s[b]).wait()

        fire_gathers(0, 0)

        def body(t, carry):
            a = 2 * t

            @pl.when(t > 0)
            def _():
                drain_scatters(1)
            fire_gathers(a + 1, 1)
            drain_gathers(0)
            fire_scatters(0)
            drain_scatters(0)

            @pl.when(t + 1 < npairs)
            def _():
                fire_gathers(a + 2, 0)
            drain_gathers(1)
            fire_scatters(1)
            return carry

        lax.fori_loop(0, npairs, body, 0)
        drain_scatters(1)
        plsc.subcore_barrier()
        dst_off = pl.multiple_of(c * NPAD + off, 8)
        pltpu.sync_copy(acc.at[pl.ds(off, NPT)], out.at[pl.ds(dst_off, NPT)])

    return agg


def _t0_body(d0_ref, d1_ref, dv_ref):
    dv_ref[...] = lax.rsqrt(d0_ref[...] + d1_ref[...] + 1.0)


def _t1_body(x_ref, dvw_ref, hs_ref):
    hs_ref[...] = x_ref[...] * dvw_ref[...]


def _t2_body(a0, a1, hs, dvw, w, b, out):
    z = (a0[...] + a1[...] + hs[...]) * dvw[...]
    u = jnp.dot(z, w[...], preferred_element_type=_F32,
                precision=lax.Precision.HIGHEST) + b[...]
    out[...] = jnp.tanh(u) * dvw[...]


def _t3_body(a0, a1, hs, dvw, w, b, s16, out):
    z = (a0[...] + a1[...] + hs[...]) * dvw[...]
    u = jnp.dot(z, w[...], preferred_element_type=_F32,
                precision=lax.Precision.HIGHEST) + b[...]
    ss = jnp.dot(u * u, s16[...], preferred_element_type=_F32,
                 precision=lax.Precision.HIGHEST)
    h = jnp.tanh(u / jnp.maximum(jnp.sqrt(ss), 1e-12))
    out[...] = h * dvw[...]


def _t4_body(a0, a1, hs, dvw, w3l, w3r, b3l, b3r, wcl, wcr, bc, s16, out):
    hp = lax.Precision.HIGHEST
    z = (a0[...] + a1[...] + hs[...]) * dvw[...]
    ul = jnp.dot(z, w3l[...], preferred_element_type=_F32, precision=hp) + b3l[...]
    ur = jnp.dot(z, w3r[...], preferred_element_type=_F32, precision=hp) + b3r[...]
    ss = jnp.dot(ul * ul + ur * ur, s16[...], preferred_element_type=_F32,
                 precision=hp)
    inv = 1.0 / jnp.maximum(jnp.sqrt(ss), 1e-12)
    h3l = ul * inv
    h3r = ur * inv
    v = (jnp.dot(h3l, wcl[...], preferred_element_type=_F32, precision=hp)
         + jnp.dot(h3r, wcr[...], preferred_element_type=_F32, precision=hp)
         + bc[...])
    ss2 = jnp.dot(v * v, s16[...], preferred_element_type=_F32, precision=hp)
    out[...] = v / jnp.maximum(jnp.sqrt(ss2), 1e-12)


def _spec(rows, imap):
    return pl.BlockSpec((rows, 128), imap)


def _cspec(shape):
    return pl.BlockSpec(shape, lambda i: (0, 0))


def _blockdiag(w16):
    return jnp.kron(jnp.eye(8, dtype=_F32), w16)


def kernel(x, edge_index, W1, b1, W2, b2, W3, b3, Wc, bc):
    n = x.shape[0]
    e = edge_index.shape[1]
    burst_edges = 32 * K * CH
    nbursts = -(-e // burst_edges)
    nbursts += nbursts % 2
    epad = nbursts * burst_edges
    erows = epad // CH
    nbursts_d = epad // (32 * KD * CH)
    assert nbursts_d * 32 * KD * CH == epad

    src = edge_index[0].astype(jnp.int32)
    dst = edge_index[1].astype(jnp.int32)
    pad = epad - e
    src2d = jnp.concatenate([src, jnp.zeros((pad,), jnp.int32)]).reshape(erows, CH)
    dst2d = jnp.concatenate([dst, jnp.full((pad,), n, jnp.int32)]).reshape(erows, CH)

    zrows = jnp.zeros((NPT, 16), _F32)
    zeros1 = jnp.zeros((NPT,), _F32)
    ones_h = jnp.ones((KD, CH), _F32)

    w1b = _blockdiag(jnp.zeros((16, 16), _F32).at[:3, :6].set(W1))
    b1b = jnp.tile(jnp.zeros((1, 16), _F32).at[0, :6].set(b1), (1, 8))
    w2b = _blockdiag(jnp.zeros((16, 16), _F32).at[:6, :12].set(W2))
    b2b = jnp.tile(jnp.zeros((1, 16), _F32).at[0, :12].set(b2), (1, 8))
    w3lb = _blockdiag(jnp.zeros((16, 16), _F32).at[:12, :12].set(W3[:, :12]))
    w3rb = _blockdiag(jnp.zeros((16, 16), _F32).at[:12, :12].set(W3[:, 12:]))
    b3lb = jnp.tile(jnp.zeros((1, 16), _F32).at[0, :12].set(b3[:12]), (1, 8))
    b3rb = jnp.tile(jnp.zeros((1, 16), _F32).at[0, :12].set(b3[12:]), (1, 8))
    wclb = _blockdiag(jnp.zeros((16, 16), _F32).at[:12, :13].set(Wc[:12]))
    wcrb = _blockdiag(jnp.zeros((16, 16), _F32).at[:12, :13].set(Wc[12:]))
    bcb = jnp.tile(jnp.zeros((1, 16), _F32).at[0, :13].set(bc), (1, 8))
    s16b = _blockdiag(jnp.ones((16, 16), _F32))

    deg = _deg_kernel(nbursts_d)(dst2d, ones_h, zeros1)
    deg2d = deg.reshape(2 * NPAD // 128, 128)

    dv_lanes = pl.pallas_call(
        _t0_body, grid=(GRID,),
        in_specs=[
            pl.BlockSpec((16, 128), lambda i: (i, 0)),
            pl.BlockSpec((16, 128), lambda i: (i + GRID, 0)),
        ],
        out_specs=pl.BlockSpec((16, 128), lambda i: (i, 0)),
        out_shape=jax.ShapeDtypeStruct((NPAD // 128, 128), _F32),
    )(deg2d, deg2d)

    dvwf = jnp.broadcast_to(dv_lanes.reshape(NPAD, 1), (NPAD, 16)).reshape(NF, 128)
    x16f = jnp.zeros((NPAD, 16), _F32).at[:n, :3].set(x).reshape(NF, 128)

    hs1f = pl.pallas_call(
        _t1_body, grid=(NF // BR,),
        in_specs=[_spec(BR, lambda i: (i, 0))] * 2,
        out_specs=_spec(BR, lambda i: (i, 0)),
        out_shape=jax.ShapeDtypeStruct((NF, 128), _F32),
    )(x16f, dvwf)

    agg = _agg_kernel(nbursts)
    nfb = NF // BR

    def dense(body, aggf, hsf, consts):
        cspecs = [_cspec(c.shape) for c in consts]
        return pl.pallas_call(
            body, grid=(nfb,),
            in_specs=[
                _spec(BR, lambda i: (i, 0)),
                _spec(BR, lambda i: (i + nfb, 0)),
                _spec(BR, lambda i: (i, 0)),
                _spec(BR, lambda i: (i, 0)),
            ] + cspecs,
            out_specs=_spec(BR, lambda i: (i, 0)),
            out_shape=jax.ShapeDtypeStruct((NF, 128), _F32),
        )(aggf, aggf, hsf, dvwf, *consts)

    eidx2d = jnp.stack([src2d, dst2d], axis=1).reshape(2 * erows, CH)

    a1f = agg(hs1f.reshape(NPAD, 16), eidx2d, zrows).reshape(2 * NF, 128)
    hs2f = dense(_t2_body, a1f, hs1f, [w1b, b1b])

    a2f = agg(hs2f.reshape(NPAD, 16), eidx2d, zrows).reshape(2 * NF, 128)
    hs3f = dense(_t3_body, a2f, hs2f, [w2b, b2b, s16b])

    a3f = agg(hs3f.reshape(NPAD, 16), eidx2d, zrows).reshape(2 * NF, 128)
    outf = dense(_t4_body, a3f, hs3f,
                 [w3lb, w3rb, b3lb, b3rb, wclb, wcrb, bcb, s16b])

    return outf.reshape(NPAD, 16)[:n, :13]
